# baseline (speedup 1.0000x reference)
import functools

import jax
import jax.numpy as jnp
from jax import lax
from jax.experimental import pallas as pl
from jax.experimental.pallas import tpu as pltpu
from jax.experimental.pallas import tpu_sc as plsc

_DIST_MIN = -10.0
_DIST_MAX = 10.0
_ATOMS = 51
_ACT = 18
_GAMMA = 0.99
_INC = (_DIST_MAX - _DIST_MIN) / (_ATOMS - 1)

_NC, _NS, _L = 2, 16, 16
_W = _NC * _NS
_CHUNK = 128


def _tc1_body(tgt_ref, rew_ref, term_ref, p_ref, bq_ref):
    R = rew_ref.shape[0]
    AW = _ACT * _ATOMS

    rr = jax.lax.broadcasted_iota(jnp.int32, (AW, 64), 0)
    cc = jax.lax.broadcasted_iota(jnp.int32, (AW, 64), 1)
    arow = rr // _ATOMS
    zrow = _DIST_MIN + (rr % _ATOMS).astype(jnp.float32) * _INC
    red = (jnp.where(cc == arow, 1.0, 0.0)
           + jnp.where(cc == arow + 32, zrow, 0.0))

    ex = jnp.exp(tgt_ref[...])
    sz = jnp.dot(ex, red, preferred_element_type=jnp.float32,
                 precision=jax.lax.Precision.HIGHEST)
    s32 = sz[:, 0:32]
    n32 = sz[:, 32:64]
    li = jax.lax.broadcasted_iota(jnp.int32, (R, 32), 1)
    q = jnp.where(li < _ACT, n32 / s32, -jnp.inf)
    qmax = jnp.max(q, axis=-1, keepdims=True)
    cand = jnp.where(q == qmax, li, 63)
    best_a = jnp.min(cand, axis=-1, keepdims=True)
    s_star = jnp.sum(jnp.where(li == best_a, s32, 0.0), axis=-1, keepdims=True)

    ab = best_a + jnp.zeros((R, _ATOMS), jnp.int32)
    terms = [jnp.where(ab == a, ex[:, a * _ATOMS:(a + 1) * _ATOMS], 0.0)
             for a in range(_ACT)]
    while len(terms) > 1:
        terms = [terms[i] + terms[i + 1] for i in range(0, len(terms) - 1, 2)] \
            + ([terms[-1]] if len(terms) % 2 else [])
    p51 = terms[0] / s_star

    kk = jax.lax.broadcasted_iota(jnp.int32, (R, _ATOMS), 1).astype(jnp.float32)
    z = _DIST_MIN + kk * _INC
    tz = jnp.clip(rew_ref[...] + (1.0 - term_ref[...]) * (z * _GAMMA),
                  _DIST_MIN, _DIST_MAX)
    bq51 = (tz - _DIST_MIN) / _INC

    pad = jnp.zeros((R, 64 - _ATOMS), jnp.float32)
    p_ref[...] = jnp.concatenate([p51, pad], axis=1)
    bq_ref[...] = jnp.concatenate([bq51, pad], axis=1)


def _sc_body(train16, act_ref, p_hbm, bq_hbm, m_hbm, sel_hbm,
             act_v, idx5_v, ofm_v, stag_v, sel_v, bq_v, p_v, m_v, sem):
    B = act_ref.shape[0]
    n16 = train16.shape[0]
    rows_per_w = B // _W
    nchunk = rows_per_w // _CHUNK
    wid = lax.axis_index("s") * _NC + lax.axis_index("c")
    lanes = lax.iota(jnp.int32, _L)
    zero16 = jnp.zeros((_L,), jnp.float32)

    def chunk_body(ci, carry):
        base = wid * rows_per_w + ci * _CHUNK
        pltpu.sync_copy(act_ref.at[pl.ds(base, _CHUNK)], act_v)
        for i in range(_CHUNK // _L):
            rl = i * _L + lanes
            a = act_v[pl.ds(i * _L, _L)]
            off = ((base + rl) * _ACT + a) * _ATOMS
            c0 = off >> 6
            ofm_v[pl.ds(i * _L, _L)] = off & 63
            plsc.store_scatter(idx5_v, [rl * 2], c0)
            plsc.store_scatter(idx5_v, [rl * 2 + 1],
                               jnp.minimum(c0 + 1, n16 - 1))
        gather = pltpu.async_copy(train16.at[idx5_v], stag_v, sem)
        pltpu.sync_copy(bq_hbm.at[pl.ds(base * 64, _CHUNK * 64)], bq_v)
        pltpu.sync_copy(p_hbm.at[pl.ds(base * 64, _CHUNK * 64)], p_v)

        def zero_body(r2, c2):
            for c in range(4):
                m_v[pl.ds(r2 * 64 + c * _L, _L)] = zero16
            return c2
        lax.fori_loop(0, _CHUNK, zero_body, 0)

        gather.wait()

        def group_body(g, c2):
            rl16 = g * _L + lanes
            flat16 = rl16 * 64
            ofm = ofm_v[pl.ds(g * _L, _L)]
            stag_base = rl16 * 128 + ofm
            sel_base = rl16 * _ATOMS
            for j in range(_ATOMS):
                bqv = plsc.load_gather(bq_v, [flat16 + j])
                pv = plsc.load_gather(p_v, [flat16 + j])
                low = bqv.astype(jnp.int32)
                f = bqv - low.astype(jnp.float32)
                stp = jnp.where(f > 0.0, 1.0, 0.0)
                up = low + stp.astype(jnp.int32)
                plsc.addupdate_scatter(m_v, [flat16 + low], pv * f)
                plsc.addupdate_scatter(m_v, [flat16 + up], pv * (stp - f))
                sq = stag_base + j
                tv = plsc.load_gather(stag_v, [sq >> 6, sq & 63])
                plsc.store_scatter(sel_v, [sel_base + j], tv)
            return c2
        lax.fori_loop(0, _CHUNK // _L, group_body, 0)

        pltpu.sync_copy(m_v, m_hbm.at[pl.ds(base * 64, _CHUNK * 64)])
        pltpu.sync_copy(sel_v, sel_hbm.at[pl.ds(base * _ATOMS, _CHUNK * _ATOMS)])
        return carry

    lax.fori_loop(0, nchunk, chunk_body, 0)


def _make_sc(B):
    return functools.partial(
        pl.kernel,
        out_type=(jax.ShapeDtypeStruct((B * 64,), jnp.float32),
                  jax.ShapeDtypeStruct((B * _ATOMS,), jnp.float32)),
        mesh=plsc.VectorSubcoreMesh(core_axis_name="c", subcore_axis_name="s"),
        compiler_params=pltpu.CompilerParams(
            needs_layout_passes=False, use_tc_tiling_on_sc=False),
        scratch_types=[
            pltpu.VMEM((_CHUNK,), jnp.int32),
            pltpu.VMEM((_CHUNK * 2,), jnp.int32),
            pltpu.VMEM((_CHUNK,), jnp.int32),
            pltpu.VMEM((_CHUNK * 2, 64), jnp.float32),
            pltpu.VMEM((_CHUNK * _ATOMS,), jnp.float32),
            pltpu.VMEM((_CHUNK * 64,), jnp.float32),
            pltpu.VMEM((_CHUNK * 64,), jnp.float32),
            pltpu.VMEM((_CHUNK * 64,), jnp.float32),
            pltpu.SemaphoreType.DMA,
        ],
    )(_sc_body)


def _tc2_body(m_ref, sel_ref, out_ref):
    cc = jax.lax.broadcasted_iota(jnp.int32, (_ATOMS, 128), 1)
    ones_col = jnp.where(cc == 0, 1.0, 0.0)
    sel = sel_ref[...]
    m51 = m_ref[:, :_ATOMS]
    e = jnp.exp(sel)
    lse = jnp.log(jnp.dot(e, ones_col, preferred_element_type=jnp.float32, precision=jax.lax.Precision.HIGHEST)[:, 0:1])
    d1 = jnp.dot(m51 * sel, ones_col, preferred_element_type=jnp.float32, precision=jax.lax.Precision.HIGHEST)[:, 0:1]
    d2 = jnp.dot(m51, ones_col, preferred_element_type=jnp.float32, precision=jax.lax.Precision.HIGHEST)[:, 0:1]
    out_ref[...] = -(d1 - lse * d2)


def kernel(training_logits, target_logits, actions, rewards, terminals):
    B = rewards.shape[0]
    R = 512
    p_sel, bq = pl.pallas_call(
        _tc1_body,
        grid=(B // R,),
        in_specs=[
            pl.BlockSpec((R, _ACT * _ATOMS), lambda i: (i, 0)),
            pl.BlockSpec((R, 1), lambda i: (i, 0)),
            pl.BlockSpec((R, 1), lambda i: (i, 0)),
        ],
        out_specs=[
            pl.BlockSpec((R, 64), lambda i: (i, 0)),
            pl.BlockSpec((R, 64), lambda i: (i, 0)),
        ],
        out_shape=[
            jax.ShapeDtypeStruct((B, 64), jnp.float32),
            jax.ShapeDtypeStruct((B, 64), jnp.float32),
        ],
    )(target_logits.reshape(B, _ACT * _ATOMS), rewards.reshape(B, 1),
      terminals.astype(jnp.float32).reshape(B, 1))

    train16 = training_logits.reshape(B * _ACT * _ATOMS // 64, 64)
    m_flat, sel_flat = _make_sc(B)(train16, actions,
                                   p_sel.reshape(B * 64), bq.reshape(B * 64))
    m = m_flat.reshape(B, 64)
    sel = sel_flat.reshape(B, _ATOMS)

    R2 = 512
    loss = pl.pallas_call(
        _tc2_body,
        grid=(B // R2,),
        in_specs=[
            pl.BlockSpec((R2, 64), lambda i: (i, 0)),
            pl.BlockSpec((R2, _ATOMS), lambda i: (i, 0)),
        ],
        out_specs=pl.BlockSpec((R2, 1), lambda i: (i, 0)),
        out_shape=jax.ShapeDtypeStruct((B, 1), jnp.float32),
    )(m, sel)
    return loss.reshape(B)


# speedup vs baseline: 1.3552x; 1.3552x over previous
import functools

import jax
import jax.numpy as jnp
from jax import lax
from jax.experimental import pallas as pl
from jax.experimental.pallas import tpu as pltpu
from jax.experimental.pallas import tpu_sc as plsc

_DIST_MIN = -10.0
_DIST_MAX = 10.0
_ATOMS = 51
_ACT = 18
_GAMMA = 0.99
_INC = (_DIST_MAX - _DIST_MIN) / (_ATOMS - 1)

_NC, _NS, _L = 2, 16, 16
_W = _NC * _NS
_CHUNK = 128


def _tc1_body(tgt_ref, rew_ref, term_ref, p_ref, bq_ref):
    R = rew_ref.shape[0]
    AW = _ACT * _ATOMS

    rr = jax.lax.broadcasted_iota(jnp.int32, (AW, 64), 0)
    cc = jax.lax.broadcasted_iota(jnp.int32, (AW, 64), 1)
    arow = rr // _ATOMS
    zrow = _DIST_MIN + (rr % _ATOMS).astype(jnp.float32) * _INC
    red = (jnp.where(cc == arow, 1.0, 0.0)
           + jnp.where(cc == arow + 32, zrow, 0.0))

    ex = jnp.exp(tgt_ref[...])
    sz = jnp.dot(ex, red, preferred_element_type=jnp.float32,
                 precision=jax.lax.Precision.HIGHEST)
    s32 = sz[:, 0:32]
    n32 = sz[:, 32:64]
    li = jax.lax.broadcasted_iota(jnp.int32, (R, 32), 1)
    q = jnp.where(li < _ACT, n32 / s32, -jnp.inf)
    qmax = jnp.max(q, axis=-1, keepdims=True)
    cand = jnp.where(q == qmax, li, 63)
    best_a = jnp.min(cand, axis=-1, keepdims=True)
    s_star = jnp.sum(jnp.where(li == best_a, s32, 0.0), axis=-1, keepdims=True)

    ab = best_a + jnp.zeros((R, _ATOMS), jnp.int32)
    terms = [jnp.where(ab == a, ex[:, a * _ATOMS:(a + 1) * _ATOMS], 0.0)
             for a in range(_ACT)]
    while len(terms) > 1:
        terms = [terms[i] + terms[i + 1] for i in range(0, len(terms) - 1, 2)] \
            + ([terms[-1]] if len(terms) % 2 else [])
    p51 = terms[0] / s_star

    kk = jax.lax.broadcasted_iota(jnp.int32, (R, _ATOMS), 1).astype(jnp.float32)
    z = _DIST_MIN + kk * _INC
    tz = jnp.clip(rew_ref[...] + (1.0 - term_ref[...]) * (z * _GAMMA),
                  _DIST_MIN, _DIST_MAX)
    bq51 = (tz - _DIST_MIN) / _INC

    pad = jnp.zeros((R, 64 - _ATOMS), jnp.float32)
    p_ref[...] = jnp.concatenate([p51, pad], axis=1)
    bq_ref[...] = jnp.concatenate([bq51, pad], axis=1)


def _sc_body(p_hbm, bq_hbm, m_hbm, bq_v, p_v, m_v):
    B = p_hbm.shape[0] // 64
    rows_per_w = B // _W
    nchunk = rows_per_w // _CHUNK
    wid = lax.axis_index("s") * _NC + lax.axis_index("c")
    lanes = lax.iota(jnp.int32, _L)
    zero16 = jnp.zeros((_L,), jnp.float32)

    def chunk_body(ci, carry):
        base = wid * rows_per_w + ci * _CHUNK
        pltpu.sync_copy(bq_hbm.at[pl.ds(base * 64, _CHUNK * 64)], bq_v)
        pltpu.sync_copy(p_hbm.at[pl.ds(base * 64, _CHUNK * 64)], p_v)

        def zero_body(r2, c2):
            for c in range(4):
                m_v[pl.ds(r2 * 64 + c * _L, _L)] = zero16
            return c2
        lax.fori_loop(0, _CHUNK, zero_body, 0)

        def group_body(g, c2):
            flat16 = (g * _L + lanes) * 64
            for j in range(_ATOMS):
                bqv = plsc.load_gather(bq_v, [flat16 + j])
                pv = plsc.load_gather(p_v, [flat16 + j])
                low = bqv.astype(jnp.int32)
                f = bqv - low.astype(jnp.float32)
                stp = jnp.where(f > 0.0, 1.0, 0.0)
                up = low + stp.astype(jnp.int32)
                plsc.addupdate_scatter(m_v, [flat16 + low], pv * f)
                plsc.addupdate_scatter(m_v, [flat16 + up], pv * (stp - f))
            return c2
        lax.fori_loop(0, _CHUNK // _L, group_body, 0)

        pltpu.sync_copy(m_v, m_hbm.at[pl.ds(base * 64, _CHUNK * 64)])
        return carry

    lax.fori_loop(0, nchunk, chunk_body, 0)


def _make_sc(B):
    return functools.partial(
        pl.kernel,
        out_type=jax.ShapeDtypeStruct((B * 64,), jnp.float32),
        mesh=plsc.VectorSubcoreMesh(core_axis_name="c", subcore_axis_name="s"),
        compiler_params=pltpu.CompilerParams(
            needs_layout_passes=False, use_tc_tiling_on_sc=False),
        scratch_types=[
            pltpu.VMEM((_CHUNK * 64,), jnp.float32),
            pltpu.VMEM((_CHUNK * 64,), jnp.float32),
            pltpu.VMEM((_CHUNK * 64,), jnp.float32),
        ],
    )(_sc_body)


def _tc2_body(m_ref, train_ref, act_ref, out_ref):
    cc = jax.lax.broadcasted_iota(jnp.int32, (_ATOMS, 128), 1)
    ones_col = jnp.where(cc == 0, 1.0, 0.0)
    act = act_ref[...]
    R = act.shape[0]
    x = train_ref[...]
    ab = act + jnp.zeros((R, _ATOMS), jnp.int32)
    terms = [jnp.where(ab == a, x[:, a * _ATOMS:(a + 1) * _ATOMS], 0.0)
             for a in range(_ACT)]
    while len(terms) > 1:
        terms = [terms[i] + terms[i + 1] for i in range(0, len(terms) - 1, 2)] \
            + ([terms[-1]] if len(terms) % 2 else [])
    sel = terms[0]
    m51 = m_ref[:, :_ATOMS]
    e = jnp.exp(sel)
    lse = jnp.log(jnp.dot(e, ones_col, preferred_element_type=jnp.float32, precision=jax.lax.Precision.HIGHEST)[:, 0:1])
    d1 = jnp.dot(m51 * sel, ones_col, preferred_element_type=jnp.float32, precision=jax.lax.Precision.HIGHEST)[:, 0:1]
    d2 = jnp.dot(m51, ones_col, preferred_element_type=jnp.float32, precision=jax.lax.Precision.HIGHEST)[:, 0:1]
    out_ref[...] = -(d1 - lse * d2)


def kernel(training_logits, target_logits, actions, rewards, terminals):
    B = rewards.shape[0]
    R = 512
    p_sel, bq = pl.pallas_call(
        _tc1_body,
        grid=(B // R,),
        in_specs=[
            pl.BlockSpec((R, _ACT * _ATOMS), lambda i: (i, 0)),
            pl.BlockSpec((R, 1), lambda i: (i, 0)),
            pl.BlockSpec((R, 1), lambda i: (i, 0)),
        ],
        out_specs=[
            pl.BlockSpec((R, 64), lambda i: (i, 0)),
            pl.BlockSpec((R, 64), lambda i: (i, 0)),
        ],
        out_shape=[
            jax.ShapeDtypeStruct((B, 64), jnp.float32),
            jax.ShapeDtypeStruct((B, 64), jnp.float32),
        ],
    )(target_logits.reshape(B, _ACT * _ATOMS), rewards.reshape(B, 1),
      terminals.astype(jnp.float32).reshape(B, 1))

    m_flat = _make_sc(B)(p_sel.reshape(B * 64), bq.reshape(B * 64))
    m = m_flat.reshape(B, 64)

    R2 = 512
    loss = pl.pallas_call(
        _tc2_body,
        grid=(B // R2,),
        in_specs=[
            pl.BlockSpec((R2, 64), lambda i: (i, 0)),
            pl.BlockSpec((R2, _ACT * _ATOMS), lambda i: (i, 0)),
            pl.BlockSpec((R2, 1), lambda i: (i, 0)),
        ],
        out_specs=pl.BlockSpec((R2, 1), lambda i: (i, 0)),
        out_shape=jax.ShapeDtypeStruct((B, 1), jnp.float32),
    )(m, training_logits.reshape(B, _ACT * _ATOMS), actions.reshape(B, 1))
    return loss.reshape(B)
